# baseline (speedup 1.0000x reference)
import functools

import jax
import jax.numpy as jnp
from jax import lax
from jax.experimental import pallas as pl
from jax.experimental.pallas import tpu as pltpu
from jax.experimental.pallas import tpu_sc as plsc

B, D, E = 1024, 2048, 64
NC, NS, L = 2, 16, 16
NW = NC * NS
BPW = B // NW
DC = 512
NDC = D // DC
SUB = 128


def _sc_body(fp_hbm, baset_hbm, deltat_hbm, out_hbm,
             base_v, out_v, fp_v, fpf_v, deltat_v):
    wid = lax.axis_index("s") * NC + lax.axis_index("c")
    b0 = wid * BPW

    pltpu.sync_copy(deltat_hbm, deltat_v)

    for dc in range(NDC):
        pltpu.sync_copy(baset_hbm.at[:, pl.ds(dc * DC, DC)], base_v)

        def b_body(bi, _, dc=dc):
            b = b0 + bi
            pltpu.sync_copy(fp_hbm.at[b, pl.ds(dc * DC, DC)], fp_v)

            def conv_body(i, _):
                off = pl.multiple_of(i * L, L)
                fpf_v[pl.ds(off, L)] = fp_v[pl.ds(off, L)].astype(jnp.float32)
                return _

            lax.fori_loop(0, DC // L, conv_body, None)

            for sub in range(DC // SUB):
                fj = [fpf_v[pl.ds(sub * SUB + j * L, L)]
                      for j in range(SUB // L)]

                def e_body(e, _, sub=sub, fj=fj):
                    dv = deltat_v[e]
                    for j in range(SUB // L):
                        off = sub * SUB + j * L
                        out_v[e, pl.ds(off, L)] = (
                            base_v[e, pl.ds(off, L)] + fj[j] * dv)
                    return _

                lax.fori_loop(0, E, e_body, None)

            pltpu.sync_copy(out_v, out_hbm.at[b, :, pl.ds(dc * DC, DC)])
            return _

        lax.fori_loop(0, BPW, b_body, None)


def kernel(fp, pair_emb, bit_emb, val_emb):
    H = D // 2
    base = (jnp.repeat(pair_emb, 2, axis=0)
            + jnp.tile(bit_emb, (H, 1))
            + val_emb[0][None, :])
    baset = base.T
    deltat = jnp.broadcast_to((val_emb[1] - val_emb[0])[:, None], (E, L))

    mesh = plsc.VectorSubcoreMesh(core_axis_name="c", subcore_axis_name="s")
    outt = pl.kernel(
        _sc_body,
        out_type=jax.ShapeDtypeStruct((B, E, D), jnp.float32),
        mesh=mesh,
        scratch_types=[
            pltpu.VMEM((E, DC), jnp.float32),
            pltpu.VMEM((E, DC), jnp.float32),
            pltpu.VMEM((DC,), jnp.int32),
            pltpu.VMEM((DC,), jnp.float32),
            pltpu.VMEM((E, L), jnp.float32),
        ],
    )(fp, baset, deltat)
    return jnp.swapaxes(outt, 1, 2)
